# Initial kernel scaffold; baseline (speedup 1.0000x reference)
import jax
import jax.numpy as jnp
from jax.experimental import pallas as pl


def kernel(tau, varhead_lookup_table):
    raise NotImplementedError("write your pallas kernel here")


# speedup vs baseline: 4.5738x; 4.5738x over previous
import functools

import jax
import jax.numpy as jnp
from jax import lax
from jax.experimental import pallas as pl
from jax.experimental.pallas import tpu as pltpu
from jax.experimental.pallas import tpu_sc as plsc

NC, NS, L = 2, 16, 16
NW = NC * NS
BATCH = 16384
TABLE_PAD = 1024
PER_W = BATCH // NW


def _softplus_body(x_ref, o_ref):
    x = x_ref[...]
    o_ref[...] = jnp.where(x > 20.0, x, jnp.log1p(jnp.exp(x)))


def _sc_gather_body(table_hbm, tau_hbm, out_hbm, table_v, idx_v, out_v):
    wid = lax.axis_index("s") * NC + lax.axis_index("c")
    base = wid * PER_W
    pltpu.sync_copy(table_hbm, table_v)
    pltpu.sync_copy(tau_hbm.at[pl.ds(base, PER_W)], idx_v)
    for k in range(PER_W // L):
        idx = idx_v[pl.ds(k * L, L)]
        out_v[pl.ds(k * L, L)] = plsc.load_gather(table_v, [idx])
    pltpu.sync_copy(out_v, out_hbm.at[pl.ds(base, PER_W)])


_sc_gather = functools.partial(
    pl.kernel,
    mesh=plsc.VectorSubcoreMesh(core_axis_name="c", subcore_axis_name="s"),
    out_type=jax.ShapeDtypeStruct((BATCH,), jnp.float32),
    scratch_types=[
        pltpu.VMEM((TABLE_PAD,), jnp.float32),
        pltpu.VMEM((PER_W,), jnp.int32),
        pltpu.VMEM((PER_W,), jnp.float32),
    ],
    compiler_params=pltpu.CompilerParams(needs_layout_passes=False),
)(_sc_gather_body)


def kernel(tau, varhead_lookup_table):
    n = varhead_lookup_table.shape[0]
    table_p = jnp.pad(varhead_lookup_table, (0, TABLE_PAD - n))
    sp = pl.pallas_call(
        _softplus_body,
        out_shape=jax.ShapeDtypeStruct((8, 128), jnp.float32),
    )(table_p.reshape(8, 128))
    return _sc_gather(sp.reshape(TABLE_PAD), tau.astype(jnp.int32))
